# Initial kernel scaffold; baseline (speedup 1.0000x reference)
import jax
import jax.numpy as jnp
from jax.experimental import pallas as pl


def kernel(z, codebook):
    raise NotImplementedError("write your pallas kernel here")


# speedup vs baseline: 10.2960x; 10.2960x over previous
import functools

import jax
import jax.numpy as jnp
from jax import lax
from jax.experimental import pallas as pl
from jax.experimental.pallas import tpu as pltpu
from jax.experimental.pallas import tpu_sc as plsc

K = 8192
D = 32
N = 16384
T = 512
NT = N // T

NC = 2
NS = 16
NW = NC * NS
BPW = N // NW
CHUNK = 128
NCHUNK = BPW // CHUNK


def _argmin_body(z_ref, cb_ref, idx_ref, loss_ref):
    z_t = z_ref[...]
    cb = cb_ref[...]
    ez = lax.dot_general(z_t, cb, dimension_numbers=(((1,), (1,)), ((), ())))
    x = z_t * z_t
    p = ((x[:, 0:8] + x[:, 8:16]) + x[:, 16:24]) + x[:, 24:32]
    b = p[:, 0:4] + p[:, 4:8]
    c = b[:, 0:2] + b[:, 2:4]
    z2 = c[:, 0:1] + c[:, 1:2]
    e2 = jnp.sum(cb * cb, axis=1)[None, :]
    d = (z2 + e2) - 2.0 * ez
    HK = K // 2
    iota = lax.broadcasted_iota(jnp.int32, (T, HK), 1)
    d0 = d[:, :HK]
    d1 = d[:, HK:]
    v0 = jnp.min(d0, axis=1, keepdims=True)
    i0 = jnp.min(jnp.where(d0 == v0, iota, K), axis=1, keepdims=True)
    v1 = jnp.min(d1, axis=1, keepdims=True)
    i1 = jnp.min(jnp.where(d1 == v1, iota + HK, K), axis=1, keepdims=True)
    q0 = v0.astype(jnp.bfloat16).astype(jnp.float32)
    keep0 = (q0 < v1) | ((q0 == v1) & (i0 < i1))
    idx = jnp.where(keep0, i0, i1)
    dsel = jnp.where(keep0, v0, v1)
    idx_ref[0, 0, :] = idx[:, 0]
    part = jnp.sum(dsel)

    @pl.when(pl.program_id(0) == 0)
    def _():
        loss_ref[0, 0] = part

    @pl.when(pl.program_id(0) != 0)
    def _():
        loss_ref[0, 0] += part


def _argmin_call(z_flat, codebook):
    return pl.pallas_call(
        _argmin_body,
        grid=(NT,),
        in_specs=[
            pl.BlockSpec((T, D), lambda i: (i, 0)),
            pl.BlockSpec((K, D), lambda i: (0, 0)),
        ],
        out_specs=[
            pl.BlockSpec((1, 1, T), lambda i: (i, 0, 0)),
            pl.BlockSpec((1, 1), lambda i: (0, 0), memory_space=pltpu.SMEM),
        ],
        out_shape=[
            jax.ShapeDtypeStruct((NT, 1, T), jnp.int32),
            jax.ShapeDtypeStruct((1, 1), jnp.float32),
        ],
    )(z_flat, codebook)


@functools.cache
def _gather_kernel():
    mesh = plsc.VectorSubcoreMesh(core_axis_name="c", subcore_axis_name="s")

    @functools.partial(
        pl.kernel,
        out_type=jax.ShapeDtypeStruct((N, D), jnp.float32),
        mesh=mesh,
        scratch_types=[
            pltpu.VMEM((NCHUNK, CHUNK), jnp.int32),
            pltpu.VMEM((BPW, D), jnp.float32),
            pltpu.SemaphoreType.DMA,
        ],
        compiler_params=pltpu.CompilerParams(use_tc_tiling_on_sc=False),
    )
    def gather(cb_hbm, idx_hbm, out_hbm, idx_v, rows_v, sem):
        wid = lax.axis_index("s") * NC + lax.axis_index("c")
        base = wid * NCHUNK
        pltpu.sync_copy(idx_hbm.at[pl.ds(base, NCHUNK)], idx_v)
        copies = [
            pltpu.async_copy(
                cb_hbm.at[idx_v.at[j]],
                rows_v.at[pl.ds(j * CHUNK, CHUNK)],
                sem,
            )
            for j in range(NCHUNK)
        ]
        for c in copies:
            c.wait()
        pltpu.sync_copy(rows_v, out_hbm.at[pl.ds(wid * BPW, BPW)])

    return gather


def _finalize_body(z_ref, zq_ref, out_ref):
    q = zq_ref[0]
    qt = lax.transpose(q, (1, 0))
    ze = z_ref[0]
    out_ref[0] = ze + (qt - ze)


def _finalize_call(z, zq3):
    B, _, L = z.shape
    return pl.pallas_call(
        _finalize_body,
        grid=(B,),
        in_specs=[
            pl.BlockSpec((1, D, L), lambda i: (i, 0, 0)),
            pl.BlockSpec((1, L, D), lambda i: (i, 0, 0)),
        ],
        out_specs=pl.BlockSpec((1, D, L), lambda i: (i, 0, 0)),
        out_shape=jax.ShapeDtypeStruct((B, D, L), jnp.float32),
    )(z, zq3)


def kernel(z, codebook):
    B, _, L = z.shape
    z_flat = jnp.transpose(z, (0, 2, 1)).reshape(N, D)
    idx, loss_sum = _argmin_call(z_flat, codebook)
    idx2d = idx.reshape(N // CHUNK, CHUNK)
    cbq = codebook.astype(jnp.bfloat16).astype(jnp.float32)
    zq_flat = _gather_kernel()(cbq, idx2d)
    z_q = _finalize_call(z, zq_flat.reshape(B, L, D))
    vq_loss = loss_sum[0, 0] * (1.25 / (N * D))
    return z_q, vq_loss


# speedup vs baseline: 11.3487x; 1.1022x over previous
import functools

import jax
import jax.numpy as jnp
from jax import lax
from jax.experimental import pallas as pl
from jax.experimental.pallas import tpu as pltpu
from jax.experimental.pallas import tpu_sc as plsc

K = 8192
D = 32
N = 16384
T = 512
NT = N // T

NC = 2
NS = 16
NW = NC * NS
BPW = N // NW
CHUNK = 128
NCHUNK = BPW // CHUNK


def _e2_body(cb_ref, e2_ref):
    cb = cb_ref[...]
    e2_ref[0, :] = jnp.sum(cb * cb, axis=1)


def _e2_call(codebook):
    return pl.pallas_call(
        _e2_body,
        out_shape=jax.ShapeDtypeStruct((1, K), jnp.float32),
    )(codebook)


def _argmin_body(z_ref, cb_ref, e2_ref, idx_ref, loss_ref):
    z_t = z_ref[...]
    cb = cb_ref[...]
    ez2 = lax.dot_general(z_t, cb + cb,
                          dimension_numbers=(((1,), (1,)), ((), ())))
    x = z_t * z_t
    p = ((x[:, 0:8] + x[:, 8:16]) + x[:, 16:24]) + x[:, 24:32]
    b = p[:, 0:4] + p[:, 4:8]
    c = b[:, 0:2] + b[:, 2:4]
    z2 = c[:, 0:1] + c[:, 1:2]
    e2 = e2_ref[...]
    d = (z2 + e2) - ez2
    HK = K // 2
    iota = lax.broadcasted_iota(jnp.int32, (T, HK), 1)
    d0 = d[:, :HK]
    d1 = d[:, HK:]
    v0 = jnp.min(d0, axis=1, keepdims=True)
    i0 = jnp.min(jnp.where(d0 == v0, iota, K), axis=1, keepdims=True)
    v1 = jnp.min(d1, axis=1, keepdims=True)
    i1 = jnp.min(jnp.where(d1 == v1, iota + HK, K), axis=1, keepdims=True)
    q0 = v0.astype(jnp.bfloat16).astype(jnp.float32)
    keep0 = (q0 < v1) | ((q0 == v1) & (i0 < i1))
    idx = jnp.where(keep0, i0, i1)
    dsel = jnp.where(keep0, v0, v1)
    idx_ref[0, 0, :] = idx[:, 0]
    part = jnp.sum(dsel)

    @pl.when(pl.program_id(0) == 0)
    def _():
        loss_ref[0, 0] = part

    @pl.when(pl.program_id(0) != 0)
    def _():
        loss_ref[0, 0] += part


def _argmin_call(z_flat, codebook, e2):
    return pl.pallas_call(
        _argmin_body,
        grid=(NT,),
        in_specs=[
            pl.BlockSpec((T, D), lambda i: (i, 0)),
            pl.BlockSpec((K, D), lambda i: (0, 0)),
            pl.BlockSpec((1, K), lambda i: (0, 0)),
        ],
        out_specs=[
            pl.BlockSpec((1, 1, T), lambda i: (i, 0, 0)),
            pl.BlockSpec((1, 1), lambda i: (0, 0), memory_space=pltpu.SMEM),
        ],
        out_shape=[
            jax.ShapeDtypeStruct((NT, 1, T), jnp.int32),
            jax.ShapeDtypeStruct((1, 1), jnp.float32),
        ],
    )(z_flat, codebook, e2)


@functools.cache
def _gather_kernel():
    mesh = plsc.VectorSubcoreMesh(core_axis_name="c", subcore_axis_name="s")

    @functools.partial(
        pl.kernel,
        out_type=jax.ShapeDtypeStruct((N, D), jnp.float32),
        mesh=mesh,
        scratch_types=[
            pltpu.VMEM((NCHUNK, CHUNK), jnp.int32),
            pltpu.VMEM((BPW, D), jnp.float32),
            pltpu.SemaphoreType.DMA,
        ],
        compiler_params=pltpu.CompilerParams(use_tc_tiling_on_sc=False),
    )
    def gather(cb_hbm, idx_hbm, out_hbm, idx_v, rows_v, sem):
        wid = lax.axis_index("s") * NC + lax.axis_index("c")
        base = wid * NCHUNK
        pltpu.sync_copy(idx_hbm.at[pl.ds(base, NCHUNK)], idx_v)
        copies = [
            pltpu.async_copy(
                cb_hbm.at[idx_v.at[j]],
                rows_v.at[pl.ds(j * CHUNK, CHUNK)],
                sem,
            )
            for j in range(NCHUNK)
        ]
        for c in copies:
            c.wait()
        pltpu.sync_copy(rows_v, out_hbm.at[pl.ds(wid * BPW, BPW)])

    return gather


def _finalize_body(z_ref, zq_ref, out_ref):
    q = zq_ref[0]
    qt = lax.transpose(q, (1, 0))
    ze = z_ref[0]
    out_ref[0] = ze + (qt - ze)


def _finalize_call(z, zq3):
    B, _, L = z.shape
    return pl.pallas_call(
        _finalize_body,
        grid=(B,),
        in_specs=[
            pl.BlockSpec((1, D, L), lambda i: (i, 0, 0)),
            pl.BlockSpec((1, L, D), lambda i: (i, 0, 0)),
        ],
        out_specs=pl.BlockSpec((1, D, L), lambda i: (i, 0, 0)),
        out_shape=jax.ShapeDtypeStruct((B, D, L), jnp.float32),
    )(z, zq3)


def kernel(z, codebook):
    B, _, L = z.shape
    z_flat = jnp.transpose(z, (0, 2, 1)).reshape(N, D)
    e2 = _e2_call(codebook)
    idx, loss_sum = _argmin_call(z_flat, codebook, e2)
    idx2d = idx.reshape(N // CHUNK, CHUNK)
    cbq = codebook.astype(jnp.bfloat16).astype(jnp.float32)
    zq_flat = _gather_kernel()(cbq, idx2d)
    z_q = _finalize_call(z, zq_flat.reshape(B, L, D))
    vq_loss = loss_sum[0, 0] * (1.25 / (N * D))
    return z_q, vq_loss
